# baseline (speedup 1.0000x reference)
import jax
import jax.numpy as jnp
from jax.experimental import pallas as pl
from jax.experimental.pallas import tpu as pltpu

_F32 = jnp.float32


def _pick_block(n, target=1024):
    best = n
    for b in range(8, min(n, target) + 1, 8):
        if n % b == 0:
            best = b
    return best if best <= target else n


def _soft_assign(p, wi1, wi2):
    t = jnp.dot(p, wi1, preferred_element_type=_F32) + p
    t = jnp.maximum(t, 0.0)
    lg = jnp.dot(t, wi2, preferred_element_type=_F32)
    m = jnp.max(lg, axis=1, keepdims=True)
    e = jnp.exp(lg - m)
    return e / jnp.sum(e, axis=1, keepdims=True)


def _pass_body(a_ref, x_ref, wi1_ref, wi2_ref, adj_ref, p_ref, hi_ref):
    r = pl.program_id(0)
    p = jnp.dot(
        a_ref[...],
        x_ref[...],
        preferred_element_type=_F32,
        precision=jax.lax.Precision.DEFAULT,
    )
    p_ref[...] = p
    h1 = _soft_assign(p, wi1_ref[...], wi2_ref[...])
    adjv = adj_ref[...]
    denom = adjv * jnp.sum(h1, axis=1, keepdims=True) + 1e-8
    b = h1 * (adjv / denom)

    @pl.when(r == 0)
    def _():
        hi_ref[...] = jnp.zeros_like(hi_ref)

    hi_ref[...] += jax.lax.dot_general(
        b, p, (((0,), (0,)), ((), ())), preferred_element_type=_F32
    )


def _adj_pass(A, X, Wi1, Wi2, adj2, br):
    n = A.shape[0]
    d = X.shape[1]
    k = Wi2.shape[1]
    return pl.pallas_call(
        _pass_body,
        grid=(n // br,),
        in_specs=[
            pl.BlockSpec((br, n), lambda r: (r, 0)),
            pl.BlockSpec((n, d), lambda r: (0, 0)),
            pl.BlockSpec((d, d), lambda r: (0, 0)),
            pl.BlockSpec((d, k), lambda r: (0, 0)),
            pl.BlockSpec((br, 1), lambda r: (r, 0)),
        ],
        out_specs=[
            pl.BlockSpec((br, d), lambda r: (r, 0)),
            pl.BlockSpec((k, d), lambda r: (0, 0)),
        ],
        out_shape=[
            jax.ShapeDtypeStruct((n, d), _F32),
            jax.ShapeDtypeStruct((k, d), _F32),
        ],
        compiler_params=pltpu.CompilerParams(dimension_semantics=("arbitrary",), vmem_limit_bytes=120 * 1024 * 1024),
    )(A, X, Wi1, Wi2, adj2)


def _xform_body(x_ref, w_ref, o_ref):
    o_ref[...] = jnp.dot(x_ref[...], w_ref[...], preferred_element_type=_F32)


def _xform(X, W, br):
    n, d = X.shape
    do = W.shape[1]
    return pl.pallas_call(
        _xform_body,
        grid=(n // br,),
        in_specs=[
            pl.BlockSpec((br, d), lambda r: (r, 0)),
            pl.BlockSpec((d, do), lambda r: (0, 0)),
        ],
        out_specs=pl.BlockSpec((br, do), lambda r: (r, 0)),
        out_shape=jax.ShapeDtypeStruct((n, do), _F32),
        compiler_params=pltpu.CompilerParams(dimension_semantics=("arbitrary",)),
    )(X, W)


def _normalize(x):
    nrm = jnp.sqrt(jnp.sum(x * x, axis=1, keepdims=True))
    return x / jnp.maximum(nrm, 1e-12)


def _tail_mid_body(p_ref, wi1_ref, wi2_ref, hi_ref, wn_ref, ni_ref, nh_ref, xn_ref):
    p = p_ref[...]
    h1 = _soft_assign(p, wi1_ref[...], wi2_ref[...])
    h = jnp.dot(h1, hi_ref[...], preferred_element_type=_F32)
    item = h + p
    ni_ref[...] = _normalize(item)
    nh_ref[...] = _normalize(h)
    xn_ref[...] = jnp.dot(item, wn_ref[...], preferred_element_type=_F32)


def _tail_mid(P, Wi1, Wi2, hi, Wn, br):
    n, d = P.shape
    k = hi.shape[0]
    return pl.pallas_call(
        _tail_mid_body,
        grid=(n // br,),
        in_specs=[
            pl.BlockSpec((br, d), lambda r: (r, 0)),
            pl.BlockSpec((d, d), lambda r: (0, 0)),
            pl.BlockSpec((d, k), lambda r: (0, 0)),
            pl.BlockSpec((k, d), lambda r: (0, 0)),
            pl.BlockSpec((d, d), lambda r: (0, 0)),
        ],
        out_specs=[pl.BlockSpec((br, d), lambda r: (r, 0))] * 3,
        out_shape=[jax.ShapeDtypeStruct((n, d), _F32)] * 3,
        compiler_params=pltpu.CompilerParams(dimension_semantics=("arbitrary",)),
    )(P, Wi1, Wi2, hi, Wn)


def _tail_last_body(p_ref, wi1_ref, wi2_ref, hi_ref, e_ref, n1_ref, g1_ref, out_ref, hs_ref):
    p = p_ref[...]
    h1 = _soft_assign(p, wi1_ref[...], wi2_ref[...])
    h = jnp.dot(h1, hi_ref[...], preferred_element_type=_F32)
    item = h + p
    out_ref[...] = (e_ref[...] + n1_ref[...] + _normalize(item)) * (1.0 / 3.0)
    hs_ref[...] = (g1_ref[...] + _normalize(h)) * 0.5


def _tail_last(P, Wi1, Wi2, hi, emb, n1, g1, br):
    n, d = P.shape
    k = hi.shape[0]
    rb = pl.BlockSpec((br, d), lambda r: (r, 0))
    return pl.pallas_call(
        _tail_last_body,
        grid=(n // br,),
        in_specs=[
            rb,
            pl.BlockSpec((d, d), lambda r: (0, 0)),
            pl.BlockSpec((d, k), lambda r: (0, 0)),
            pl.BlockSpec((k, d), lambda r: (0, 0)),
            rb,
            rb,
            rb,
        ],
        out_specs=[rb, rb],
        out_shape=[jax.ShapeDtypeStruct((n, d), _F32)] * 2,
        compiler_params=pltpu.CompilerParams(dimension_semantics=("arbitrary",)),
    )(P, Wi1, Wi2, hi, emb, n1, g1)


def kernel(adj, adjacency, embedding, W_item0, W_item1, W_i1, W_i2, channel):
    n, d = embedding.shape
    br = _pick_block(n, 2000)
    sbr = _pick_block(n, 500)
    adj2 = adj.reshape(n, 1)

    x1 = _xform(embedding, W_item0, br)
    p1, hi_1 = _adj_pass(adjacency, x1, W_i1, W_i2, adj2, sbr)
    n1, g1, x2 = _tail_mid(p1, W_i1, W_i2, hi_1, W_item1, br)

    p2, hi_2 = _adj_pass(adjacency, x2, W_i1, W_i2, adj2, sbr)
    out, hs = _tail_last(p2, W_i1, W_i2, hi_2, embedding, n1, g1, br)
    return (out, hs)


# speedup vs baseline: 1.0621x; 1.0621x over previous
import jax
import jax.numpy as jnp
from jax.experimental import pallas as pl
from jax.experimental.pallas import tpu as pltpu

_F32 = jnp.float32


def _pick_block(n, target=1024):
    best = n
    for b in range(8, min(n, target) + 1, 8):
        if n % b == 0:
            best = b
    return best if best <= target else n


def _soft_assign(p, wi1, wi2):
    t = jnp.dot(p, wi1, preferred_element_type=_F32) + p
    t = jnp.maximum(t, 0.0)
    lg = jnp.dot(t, wi2, preferred_element_type=_F32)
    m = jnp.max(lg, axis=1, keepdims=True)
    e = jnp.exp(lg - m)
    return e / jnp.sum(e, axis=1, keepdims=True)


def _pass1_body(
    a_ref, e_ref, w0_ref, wi1_ref, wi2_ref, adj_ref, p_ref, hi_ref, x_ref
):
    r = pl.program_id(0)

    @pl.when(r == 0)
    def _():
        x_ref[...] = jnp.dot(e_ref[...], w0_ref[...], preferred_element_type=_F32)

    p = jnp.dot(a_ref[...], x_ref[...], preferred_element_type=_F32)
    p_ref[...] = p
    h1 = _soft_assign(p, wi1_ref[...], wi2_ref[...])
    adjv = adj_ref[...]
    denom = adjv * jnp.sum(h1, axis=1, keepdims=True) + 1e-8
    b = h1 * (adjv / denom)

    @pl.when(r == 0)
    def _():
        hi_ref[...] = jnp.zeros_like(hi_ref)

    hi_ref[...] += jax.lax.dot_general(
        b, p, (((0,), (0,)), ((), ())), preferred_element_type=_F32
    )


def _adj_pass1(A, emb, W0, Wi1, Wi2, adj2, br):
    n = A.shape[0]
    d = emb.shape[1]
    k = Wi2.shape[1]
    return pl.pallas_call(
        _pass1_body,
        grid=(n // br,),
        in_specs=[
            pl.BlockSpec((br, n), lambda r: (r, 0)),
            pl.BlockSpec((n, d), lambda r: (0, 0)),
            pl.BlockSpec((d, d), lambda r: (0, 0)),
            pl.BlockSpec((d, d), lambda r: (0, 0)),
            pl.BlockSpec((d, k), lambda r: (0, 0)),
            pl.BlockSpec((br, 1), lambda r: (r, 0)),
        ],
        out_specs=[
            pl.BlockSpec((br, d), lambda r: (r, 0)),
            pl.BlockSpec((k, d), lambda r: (0, 0)),
        ],
        out_shape=[
            jax.ShapeDtypeStruct((n, d), _F32),
            jax.ShapeDtypeStruct((k, d), _F32),
        ],
        scratch_shapes=[pltpu.VMEM((n, d), _F32)],
        compiler_params=pltpu.CompilerParams(dimension_semantics=("arbitrary",)),
    )(A, emb, W0, Wi1, Wi2, adj2)


def _pass2_body(a_ref, x_ref, wi1_ref, wi2_ref, adj_ref, p_ref, hi_ref):
    r = pl.program_id(0)
    p = jnp.dot(a_ref[...], x_ref[...], preferred_element_type=_F32)
    p_ref[...] = p
    h1 = _soft_assign(p, wi1_ref[...], wi2_ref[...])
    adjv = adj_ref[...]
    denom = adjv * jnp.sum(h1, axis=1, keepdims=True) + 1e-8
    b = h1 * (adjv / denom)

    @pl.when(r == 0)
    def _():
        hi_ref[...] = jnp.zeros_like(hi_ref)

    hi_ref[...] += jax.lax.dot_general(
        b, p, (((0,), (0,)), ((), ())), preferred_element_type=_F32
    )


def _adj_pass2(A, X, Wi1, Wi2, adj2, br):
    n = A.shape[0]
    d = X.shape[1]
    k = Wi2.shape[1]
    return pl.pallas_call(
        _pass2_body,
        grid=(n // br,),
        in_specs=[
            pl.BlockSpec((br, n), lambda r: (r, 0)),
            pl.BlockSpec((n, d), lambda r: (0, 0)),
            pl.BlockSpec((d, d), lambda r: (0, 0)),
            pl.BlockSpec((d, k), lambda r: (0, 0)),
            pl.BlockSpec((br, 1), lambda r: (r, 0)),
        ],
        out_specs=[
            pl.BlockSpec((br, d), lambda r: (r, 0)),
            pl.BlockSpec((k, d), lambda r: (0, 0)),
        ],
        out_shape=[
            jax.ShapeDtypeStruct((n, d), _F32),
            jax.ShapeDtypeStruct((k, d), _F32),
        ],
        compiler_params=pltpu.CompilerParams(dimension_semantics=("arbitrary",)),
    )(A, X, Wi1, Wi2, adj2)


def _normalize(x):
    nrm = jnp.sqrt(jnp.sum(x * x, axis=1, keepdims=True))
    return x / jnp.maximum(nrm, 1e-12)


def _tail_mid_body(p_ref, wi1_ref, wi2_ref, hi_ref, wn_ref, ni_ref, nh_ref, xn_ref):
    p = p_ref[...]
    h1 = _soft_assign(p, wi1_ref[...], wi2_ref[...])
    h = jnp.dot(h1, hi_ref[...], preferred_element_type=_F32)
    item = h + p
    ni_ref[...] = _normalize(item)
    nh_ref[...] = _normalize(h)
    xn_ref[...] = jnp.dot(item, wn_ref[...], preferred_element_type=_F32)


def _tail_mid(P, Wi1, Wi2, hi, Wn, br):
    n, d = P.shape
    k = hi.shape[0]
    return pl.pallas_call(
        _tail_mid_body,
        grid=(n // br,),
        in_specs=[
            pl.BlockSpec((br, d), lambda r: (r, 0)),
            pl.BlockSpec((d, d), lambda r: (0, 0)),
            pl.BlockSpec((d, k), lambda r: (0, 0)),
            pl.BlockSpec((k, d), lambda r: (0, 0)),
            pl.BlockSpec((d, d), lambda r: (0, 0)),
        ],
        out_specs=[pl.BlockSpec((br, d), lambda r: (r, 0))] * 3,
        out_shape=[jax.ShapeDtypeStruct((n, d), _F32)] * 3,
        compiler_params=pltpu.CompilerParams(dimension_semantics=("arbitrary",)),
    )(P, Wi1, Wi2, hi, Wn)


def _tail_last_body(p_ref, wi1_ref, wi2_ref, hi_ref, e_ref, n1_ref, g1_ref, out_ref, hs_ref):
    p = p_ref[...]
    h1 = _soft_assign(p, wi1_ref[...], wi2_ref[...])
    h = jnp.dot(h1, hi_ref[...], preferred_element_type=_F32)
    item = h + p
    out_ref[...] = (e_ref[...] + n1_ref[...] + _normalize(item)) * (1.0 / 3.0)
    hs_ref[...] = (g1_ref[...] + _normalize(h)) * 0.5


def _tail_last(P, Wi1, Wi2, hi, emb, n1, g1, br):
    n, d = P.shape
    k = hi.shape[0]
    rb = pl.BlockSpec((br, d), lambda r: (r, 0))
    return pl.pallas_call(
        _tail_last_body,
        grid=(n // br,),
        in_specs=[
            rb,
            pl.BlockSpec((d, d), lambda r: (0, 0)),
            pl.BlockSpec((d, k), lambda r: (0, 0)),
            pl.BlockSpec((k, d), lambda r: (0, 0)),
            rb,
            rb,
            rb,
        ],
        out_specs=[rb, rb],
        out_shape=[jax.ShapeDtypeStruct((n, d), _F32)] * 2,
        compiler_params=pltpu.CompilerParams(dimension_semantics=("arbitrary",)),
    )(P, Wi1, Wi2, hi, emb, n1, g1)


def kernel(adj, adjacency, embedding, W_item0, W_item1, W_i1, W_i2, channel):
    n, d = embedding.shape
    br = _pick_block(n, 2000)
    sbr = _pick_block(n, 500)
    adj2 = adj.reshape(n, 1)

    p1, hi_1 = _adj_pass1(adjacency, embedding, W_item0, W_i1, W_i2, adj2, sbr)
    n1, g1, x2 = _tail_mid(p1, W_i1, W_i2, hi_1, W_item1, br)

    p2, hi_2 = _adj_pass2(adjacency, x2, W_i1, W_i2, adj2, sbr)
    out, hs = _tail_last(p2, W_i1, W_i2, hi_2, embedding, n1, g1, br)
    return (out, hs)


# speedup vs baseline: 1.0703x; 1.0078x over previous
import jax
import jax.numpy as jnp
from jax.experimental import pallas as pl
from jax.experimental.pallas import tpu as pltpu

_F32 = jnp.float32


def _pick_block(n, target=1024):
    best = n
    for b in range(8, min(n, target) + 1, 8):
        if n % b == 0:
            best = b
    return best if best <= target else n


def _soft_assign(p, wi1, wi2):
    t = jnp.dot(p, wi1, preferred_element_type=_F32) + p
    t = jnp.maximum(t, 0.0)
    lg = jnp.dot(t, wi2, preferred_element_type=_F32)
    m = jnp.max(lg, axis=1, keepdims=True)
    e = jnp.exp(lg - m)
    return e / jnp.sum(e, axis=1, keepdims=True)


def _acc_tail(p, wi1_ref, wi2_ref, adj_ref, p_ref, hi_ref, r):
    p_ref[...] = p
    h1 = _soft_assign(p, wi1_ref[...], wi2_ref[...])
    adjv = adj_ref[...]
    denom = adjv * jnp.sum(h1, axis=1, keepdims=True) + 1e-8
    b = h1 * (adjv / denom)

    @pl.when(r == 0)
    def _():
        hi_ref[...] = jnp.zeros_like(hi_ref)

    hi_ref[...] += jax.lax.dot_general(
        b, p, (((0,), (0,)), ((), ())), preferred_element_type=_F32
    )


def _start_fetch(a_hbm, step, br, b0, b1, s0, s1, nsteps):
    @pl.when(jnp.logical_and(step < nsteps, step % 2 == 0))
    def _():
        pltpu.make_async_copy(
            a_hbm.at[pl.ds(step * br, br), :], b0, s0
        ).start()

    @pl.when(jnp.logical_and(step < nsteps, step % 2 == 1))
    def _():
        pltpu.make_async_copy(
            a_hbm.at[pl.ds(step * br, br), :], b1, s1
        ).start()


def _pass1_body(
    a_hbm, e_ref, w0_ref, wi1_ref, wi2_ref, adj_ref, p_ref, hi_ref,
    x_ref, b0, b1, s0, s1
):
    r = pl.program_id(0)
    nsteps = pl.num_programs(0)
    br = b0.shape[0]

    @pl.when(r == 0)
    def _():
        pltpu.make_async_copy(a_hbm.at[pl.ds(0, br), :], b0, s0).start()
        x_ref[...] = jnp.dot(e_ref[...], w0_ref[...], preferred_element_type=_F32)

    _start_fetch(a_hbm, r + 1, br, b0, b1, s0, s1, nsteps)

    @pl.when(r % 2 == 0)
    def _():
        pltpu.make_async_copy(a_hbm.at[pl.ds(r * br, br), :], b0, s0).wait()
        p = jnp.dot(b0[...], x_ref[...], preferred_element_type=_F32)
        _acc_tail(p, wi1_ref, wi2_ref, adj_ref, p_ref, hi_ref, r)

    @pl.when(r % 2 == 1)
    def _():
        pltpu.make_async_copy(a_hbm.at[pl.ds(r * br, br), :], b1, s1).wait()
        p = jnp.dot(b1[...], x_ref[...], preferred_element_type=_F32)
        _acc_tail(p, wi1_ref, wi2_ref, adj_ref, p_ref, hi_ref, r)


def _adj_pass1(A, emb, W0, Wi1, Wi2, adj2, br):
    n = A.shape[0]
    d = emb.shape[1]
    k = Wi2.shape[1]
    return pl.pallas_call(
        _pass1_body,
        grid=(n // br,),
        in_specs=[
            pl.BlockSpec(memory_space=pl.ANY),
            pl.BlockSpec((n, d), lambda r: (0, 0)),
            pl.BlockSpec((d, d), lambda r: (0, 0)),
            pl.BlockSpec((d, d), lambda r: (0, 0)),
            pl.BlockSpec((d, k), lambda r: (0, 0)),
            pl.BlockSpec((br, 1), lambda r: (r, 0)),
        ],
        out_specs=[
            pl.BlockSpec((br, d), lambda r: (r, 0)),
            pl.BlockSpec((k, d), lambda r: (0, 0)),
        ],
        out_shape=[
            jax.ShapeDtypeStruct((n, d), _F32),
            jax.ShapeDtypeStruct((k, d), _F32),
        ],
        scratch_shapes=[
            pltpu.VMEM((n, d), _F32),
            pltpu.VMEM((br, n), _F32),
            pltpu.VMEM((br, n), _F32),
            pltpu.SemaphoreType.DMA,
            pltpu.SemaphoreType.DMA,
        ],
        compiler_params=pltpu.CompilerParams(dimension_semantics=("arbitrary",)),
    )(A, emb, W0, Wi1, Wi2, adj2)


def _pass2_body(
    a_hbm, x_ref, wi1_ref, wi2_ref, adj_ref, p_ref, hi_ref, b0, b1, s0, s1
):
    r = pl.program_id(0)
    nsteps = pl.num_programs(0)
    br = b0.shape[0]

    @pl.when(r == 0)
    def _():
        pltpu.make_async_copy(a_hbm.at[pl.ds(0, br), :], b0, s0).start()

    _start_fetch(a_hbm, r + 1, br, b0, b1, s0, s1, nsteps)

    @pl.when(r % 2 == 0)
    def _():
        pltpu.make_async_copy(a_hbm.at[pl.ds(r * br, br), :], b0, s0).wait()
        p = jnp.dot(b0[...], x_ref[...], preferred_element_type=_F32)
        _acc_tail(p, wi1_ref, wi2_ref, adj_ref, p_ref, hi_ref, r)

    @pl.when(r % 2 == 1)
    def _():
        pltpu.make_async_copy(a_hbm.at[pl.ds(r * br, br), :], b1, s1).wait()
        p = jnp.dot(b1[...], x_ref[...], preferred_element_type=_F32)
        _acc_tail(p, wi1_ref, wi2_ref, adj_ref, p_ref, hi_ref, r)


def _adj_pass2(A, X, Wi1, Wi2, adj2, br):
    n = A.shape[0]
    d = X.shape[1]
    k = Wi2.shape[1]
    return pl.pallas_call(
        _pass2_body,
        grid=(n // br,),
        in_specs=[
            pl.BlockSpec(memory_space=pl.ANY),
            pl.BlockSpec((n, d), lambda r: (0, 0)),
            pl.BlockSpec((d, d), lambda r: (0, 0)),
            pl.BlockSpec((d, k), lambda r: (0, 0)),
            pl.BlockSpec((br, 1), lambda r: (r, 0)),
        ],
        out_specs=[
            pl.BlockSpec((br, d), lambda r: (r, 0)),
            pl.BlockSpec((k, d), lambda r: (0, 0)),
        ],
        out_shape=[
            jax.ShapeDtypeStruct((n, d), _F32),
            jax.ShapeDtypeStruct((k, d), _F32),
        ],
        scratch_shapes=[
            pltpu.VMEM((br, n), _F32),
            pltpu.VMEM((br, n), _F32),
            pltpu.SemaphoreType.DMA,
            pltpu.SemaphoreType.DMA,
        ],
        compiler_params=pltpu.CompilerParams(dimension_semantics=("arbitrary",)),
    )(A, X, Wi1, Wi2, adj2)


def _normalize(x):
    nrm = jnp.sqrt(jnp.sum(x * x, axis=1, keepdims=True))
    return x / jnp.maximum(nrm, 1e-12)


def _tail_mid_body(p_ref, wi1_ref, wi2_ref, hi_ref, wn_ref, ni_ref, nh_ref, xn_ref):
    p = p_ref[...]
    h1 = _soft_assign(p, wi1_ref[...], wi2_ref[...])
    h = jnp.dot(h1, hi_ref[...], preferred_element_type=_F32)
    item = h + p
    ni_ref[...] = _normalize(item)
    nh_ref[...] = _normalize(h)
    xn_ref[...] = jnp.dot(item, wn_ref[...], preferred_element_type=_F32)


def _tail_mid(P, Wi1, Wi2, hi, Wn, br):
    n, d = P.shape
    k = hi.shape[0]
    return pl.pallas_call(
        _tail_mid_body,
        grid=(n // br,),
        in_specs=[
            pl.BlockSpec((br, d), lambda r: (r, 0)),
            pl.BlockSpec((d, d), lambda r: (0, 0)),
            pl.BlockSpec((d, k), lambda r: (0, 0)),
            pl.BlockSpec((k, d), lambda r: (0, 0)),
            pl.BlockSpec((d, d), lambda r: (0, 0)),
        ],
        out_specs=[pl.BlockSpec((br, d), lambda r: (r, 0))] * 3,
        out_shape=[jax.ShapeDtypeStruct((n, d), _F32)] * 3,
        compiler_params=pltpu.CompilerParams(dimension_semantics=("arbitrary",)),
    )(P, Wi1, Wi2, hi, Wn)


def _tail_last_body(p_ref, wi1_ref, wi2_ref, hi_ref, e_ref, n1_ref, g1_ref, out_ref, hs_ref):
    p = p_ref[...]
    h1 = _soft_assign(p, wi1_ref[...], wi2_ref[...])
    h = jnp.dot(h1, hi_ref[...], preferred_element_type=_F32)
    item = h + p
    out_ref[...] = (e_ref[...] + n1_ref[...] + _normalize(item)) * (1.0 / 3.0)
    hs_ref[...] = (g1_ref[...] + _normalize(h)) * 0.5


def _tail_last(P, Wi1, Wi2, hi, emb, n1, g1, br):
    n, d = P.shape
    k = hi.shape[0]
    rb = pl.BlockSpec((br, d), lambda r: (r, 0))
    return pl.pallas_call(
        _tail_last_body,
        grid=(n // br,),
        in_specs=[
            rb,
            pl.BlockSpec((d, d), lambda r: (0, 0)),
            pl.BlockSpec((d, k), lambda r: (0, 0)),
            pl.BlockSpec((k, d), lambda r: (0, 0)),
            rb,
            rb,
            rb,
        ],
        out_specs=[rb, rb],
        out_shape=[jax.ShapeDtypeStruct((n, d), _F32)] * 2,
        compiler_params=pltpu.CompilerParams(dimension_semantics=("arbitrary",)),
    )(P, Wi1, Wi2, hi, emb, n1, g1)


def kernel(adj, adjacency, embedding, W_item0, W_item1, W_i1, W_i2, channel):
    n, d = embedding.shape
    br = _pick_block(n, 2000)
    sbr = _pick_block(n, 500)
    adj2 = adj.reshape(n, 1)

    p1, hi_1 = _adj_pass1(adjacency, embedding, W_item0, W_i1, W_i2, adj2, sbr)
    n1, g1, x2 = _tail_mid(p1, W_i1, W_i2, hi_1, W_item1, br)

    p2, hi_2 = _adj_pass2(adjacency, x2, W_i1, W_i2, adj2, sbr)
    out, hs = _tail_last(p2, W_i1, W_i2, hi_2, embedding, n1, g1, br)
    return (out, hs)
